# baseline (speedup 1.0000x reference)
import functools

import jax
import jax.numpy as jnp
from jax import lax
from jax.experimental import pallas as pl
from jax.experimental.pallas import tpu as pltpu
from jax.experimental.pallas import tpu_sc as plsc

_B, _T, _D = 4, 4096, 2048
_N = _B * _T
_NE = 16
_NSCORE = 24
_THRESH = 0.15
_NEG = -1e30
_NW = 32
_CHUNK = _N // _NW
_NG = _CHUNK // 16


def _scores_body(x_ref, w_ref, st_ref):
    s = jax.lax.dot_general(
        x_ref[...], w_ref[...], (((1,), (0,)), ((), ())),
        preferred_element_type=jnp.float32)
    st_ref[...] = s.T


def _route_body(st_hbm, rw_hbm, idx_hbm, cnt_hbm, sv, rwv, idxv, cntv):
    wid = lax.axis_index("s") * 2 + lax.axis_index("c")
    base = wid * _CHUNK
    pltpu.sync_copy(st_hbm.at[:, pl.ds(base, _CHUNK)], sv)
    cntv[...] = jnp.zeros((16,), jnp.float32)
    iota = lax.iota(jnp.int32, 16)
    zf = jnp.zeros((16,), jnp.float32)
    zi = jnp.zeros((16,), jnp.int32)
    ones = jnp.ones((16,), jnp.float32)

    def group(g, pos):
        col = g * 16
        s = [sv[e, pl.ds(col, 16)] for e in range(18)]
        sig0 = 1.0 / (1.0 + jnp.exp(-s[16]))
        sig1 = 1.0 / (1.0 + jnp.exp(-s[17]))

        def top1(lo, hi):
            m = s[lo]
            for e in range(lo + 1, hi):
                m = jnp.maximum(m, s[e])
            idx = jnp.full((16,), hi - 1, jnp.int32)
            for e in range(hi - 2, lo - 1, -1):
                idx = jnp.where(s[e] == m, e, idx)
            z = zf
            for e in range(lo, hi):
                z = z + jnp.exp(s[e] - m)
            return m, idx, z

        _, idx_a, z_a = top1(0, 8)
        w0 = 1.0 / z_a

        _, idx_b, z_b = top1(8, 12)
        w1 = (1.0 / z_b) * sig0 * (sig0 > _THRESH).astype(jnp.float32)

        m_c, idx_c1, z_c = top1(12, 16)
        s2 = [jnp.where(idx_c1 == e, _NEG, s[e]) for e in range(12, 16)]
        m2 = jnp.maximum(jnp.maximum(s2[0], s2[1]), jnp.maximum(s2[2], s2[3]))
        idx_c2 = jnp.full((16,), 15, jnp.int32)
        for e in range(14, 11, -1):
            idx_c2 = jnp.where(s2[e - 12] == m2, e, idx_c2)
        gate_c = sig1 * (sig1 > _THRESH).astype(jnp.float32)
        w2 = (1.0 / z_c) * gate_c
        w3 = (jnp.exp(m2 - m_c) / z_c) * gate_c

        rnorm = 1.0 / (w0 + w1 + w2 + w3 + 1e-8)
        plsc.store_scatter(rwv, [pos], w0 * rnorm)
        plsc.store_scatter(rwv, [pos + 1], w1 * rnorm)
        plsc.store_scatter(rwv, [pos + 2], w2 * rnorm)
        plsc.store_scatter(rwv, [pos + 3], w3 * rnorm)
        plsc.store_scatter(rwv, [pos + 4], zf)
        plsc.store_scatter(rwv, [pos + 5], zf)
        plsc.store_scatter(idxv, [pos], idx_a)
        plsc.store_scatter(idxv, [pos + 1], idx_b)
        plsc.store_scatter(idxv, [pos + 2], idx_c1)
        plsc.store_scatter(idxv, [pos + 3], idx_c2)
        plsc.store_scatter(idxv, [pos + 4], zi)
        plsc.store_scatter(idxv, [pos + 5], zi)
        plsc.addupdate_scatter(cntv, [idx_a], ones)
        plsc.addupdate_scatter(cntv, [idx_b], ones)
        plsc.addupdate_scatter(cntv, [idx_c1], ones)
        plsc.addupdate_scatter(cntv, [idx_c2], ones)
        return pos + 96

    lax.fori_loop(0, _NG, group, iota * 6)
    pltpu.sync_copy(rwv, rw_hbm.at[pl.ds(base * 6, _CHUNK * 6)])
    pltpu.sync_copy(idxv, idx_hbm.at[pl.ds(base * 6, _CHUNK * 6)])
    pltpu.sync_copy(cntv, cnt_hbm.at[wid])


def _aux_body(cnt_ref, aux_ref):
    c = jnp.sum(cnt_ref[...], axis=0, keepdims=True)
    lane = lax.broadcasted_iota(jnp.int32, (1, _NE), 1)
    c = c + jnp.where(lane == 0, jnp.float32(2 * _B * _T), 0.0)
    total = jnp.sum(c)
    aux = (0.01 / _NE) * jnp.sum(
        jnp.log(jnp.float32(1.0 / _NE)) - jnp.log(c / total),
        axis=-1, keepdims=True)
    aux_ref[...] = aux


@functools.partial(jax.jit, static_argnames=("tb",))
def _run(x, W_expert, W_group, tb=512):
    nblocks = _N // tb
    xf = x.reshape(_N, _D)
    w18 = jnp.concatenate([W_expert, W_group], axis=0)
    wt = jnp.pad(w18, ((0, _NSCORE - 18), (0, 0))).T

    scores_t = pl.pallas_call(
        _scores_body,
        grid=(nblocks,),
        in_specs=[
            pl.BlockSpec((tb, _D), lambda i: (i, 0)),
            pl.BlockSpec((_D, _NSCORE), lambda i: (0, 0)),
        ],
        out_specs=pl.BlockSpec((_NSCORE, tb), lambda i: (0, i)),
        out_shape=jax.ShapeDtypeStruct((_NSCORE, _N), jnp.float32),
        compiler_params=pltpu.CompilerParams(
            dimension_semantics=("arbitrary",)),
    )(xf, wt)

    route = functools.partial(
        pl.kernel,
        out_type=[
            jax.ShapeDtypeStruct((_N * 6,), jnp.float32),
            jax.ShapeDtypeStruct((_N * 6,), jnp.int32),
            jax.ShapeDtypeStruct((_NW, _NE), jnp.float32),
        ],
        mesh=plsc.VectorSubcoreMesh(core_axis_name="c", subcore_axis_name="s"),
        scratch_types=[
            pltpu.VMEM((_NSCORE, _CHUNK), jnp.float32),
            pltpu.VMEM((_CHUNK * 6,), jnp.float32),
            pltpu.VMEM((_CHUNK * 6,), jnp.int32),
            pltpu.VMEM((16,), jnp.float32),
        ],
        compiler_params=pltpu.CompilerParams(needs_layout_passes=False),
    )(_route_body)
    rw_flat, idx_flat, counts = route(scores_t)

    aux = pl.pallas_call(
        _aux_body,
        out_shape=jax.ShapeDtypeStruct((1, 1), jnp.float32),
    )(counts)

    return (rw_flat.reshape(_B, _T, 6), idx_flat.reshape(_B, _T, 6),
            aux[0, 0])


def kernel(x, W_expert, W_group):
    return _run(x, W_expert, W_group)


# speedup vs baseline: 1.0734x; 1.0734x over previous
import functools

import jax
import jax.numpy as jnp
from jax import lax
from jax.experimental import pallas as pl
from jax.experimental.pallas import tpu as pltpu
from jax.experimental.pallas import tpu_sc as plsc

_B, _T, _D = 4, 4096, 2048
_N = _B * _T
_NE = 16
_NSCORE = 24
_THRESH = 0.15
_NEG = -1e30
_NW = 32
_CHUNK = _N // _NW
_NG = _CHUNK // 16


def _scores_body(x_ref, w_ref, st_ref):
    s = jax.lax.dot_general(
        x_ref[...], w_ref[...], (((1,), (0,)), ((), ())),
        preferred_element_type=jnp.float32)
    st_ref[...] = s.T


def _route_body(st_hbm, rw_hbm, idx_hbm, cnt_hbm, sv, rwv, idxv, cntv):
    wid = lax.axis_index("s") * 2 + lax.axis_index("c")
    base = wid * _CHUNK
    pltpu.sync_copy(st_hbm.at[:, pl.ds(base, _CHUNK)], sv)
    cntv[...] = jnp.zeros((16,), jnp.float32)
    iota = lax.iota(jnp.int32, 16)
    zf = jnp.zeros((16,), jnp.float32)
    zi = jnp.zeros((16,), jnp.int32)
    ones = jnp.ones((16,), jnp.float32)

    def group(g, pos):
        col = g * 16
        s = [sv[e, pl.ds(col, 16)] for e in range(18)]
        sig0 = 1.0 / (1.0 + jnp.exp(-s[16]))
        sig1 = 1.0 / (1.0 + jnp.exp(-s[17]))

        def top1(lo, hi):
            m = s[lo]
            for e in range(lo + 1, hi):
                m = jnp.maximum(m, s[e])
            idx = jnp.full((16,), hi - 1, jnp.int32)
            for e in range(hi - 2, lo - 1, -1):
                idx = jnp.where(s[e] == m, e, idx)
            z = zf
            for e in range(lo, hi):
                z = z + jnp.exp(s[e] - m)
            return m, idx, z

        _, idx_a, z_a = top1(0, 8)
        w0 = 1.0 / z_a

        _, idx_b, z_b = top1(8, 12)
        w1 = (1.0 / z_b) * sig0 * (sig0 > _THRESH).astype(jnp.float32)

        m_c, idx_c1, z_c = top1(12, 16)
        s2 = [jnp.where(idx_c1 == e, _NEG, s[e]) for e in range(12, 16)]
        m2 = jnp.maximum(jnp.maximum(s2[0], s2[1]), jnp.maximum(s2[2], s2[3]))
        idx_c2 = jnp.full((16,), 15, jnp.int32)
        for e in range(14, 11, -1):
            idx_c2 = jnp.where(s2[e - 12] == m2, e, idx_c2)
        gate_c = sig1 * (sig1 > _THRESH).astype(jnp.float32)
        w2 = (1.0 / z_c) * gate_c
        w3 = (jnp.exp(m2 - m_c) / z_c) * gate_c

        rnorm = 1.0 / (w0 + w1 + w2 + w3 + 1e-8)
        plsc.store_scatter(rwv, [pos], w0 * rnorm)
        plsc.store_scatter(rwv, [pos + 1], w1 * rnorm)
        plsc.store_scatter(rwv, [pos + 2], w2 * rnorm)
        plsc.store_scatter(rwv, [pos + 3], w3 * rnorm)
        plsc.store_scatter(rwv, [pos + 4], zf)
        plsc.store_scatter(rwv, [pos + 5], zf)
        plsc.store_scatter(idxv, [pos], idx_a)
        plsc.store_scatter(idxv, [pos + 1], idx_b)
        plsc.store_scatter(idxv, [pos + 2], idx_c1)
        plsc.store_scatter(idxv, [pos + 3], idx_c2)
        plsc.store_scatter(idxv, [pos + 4], zi)
        plsc.store_scatter(idxv, [pos + 5], zi)
        plsc.addupdate_scatter(cntv, [idx_a], ones)
        plsc.addupdate_scatter(cntv, [idx_b], ones)
        plsc.addupdate_scatter(cntv, [idx_c1], ones)
        plsc.addupdate_scatter(cntv, [idx_c2], ones)
        return pos + 96

    lax.fori_loop(0, _NG, group, iota * 6)
    pltpu.sync_copy(rwv, rw_hbm.at[pl.ds(base * 6, _CHUNK * 6)])
    pltpu.sync_copy(idxv, idx_hbm.at[pl.ds(base * 6, _CHUNK * 6)])
    pltpu.sync_copy(cntv, cnt_hbm.at[wid])


def _aux_body(cnt_ref, aux_ref):
    c = jnp.sum(cnt_ref[...], axis=0, keepdims=True)
    lane = lax.broadcasted_iota(jnp.int32, (1, _NE), 1)
    c = c + jnp.where(lane == 0, jnp.float32(2 * _B * _T), 0.0)
    total = jnp.sum(c)
    aux = (0.01 / _NE) * jnp.sum(
        jnp.log(jnp.float32(1.0 / _NE)) - jnp.log(c / total),
        axis=-1, keepdims=True)
    aux_ref[...] = aux


@functools.partial(jax.jit, static_argnames=("tb",))
def _run(x, W_expert, W_group, tb=1024):
    nblocks = _N // tb
    xf = x.reshape(_N, _D)
    w18 = jnp.concatenate([W_expert, W_group], axis=0)
    wt = jnp.pad(w18, ((0, _NSCORE - 18), (0, 0))).T

    scores_t = pl.pallas_call(
        _scores_body,
        grid=(nblocks,),
        in_specs=[
            pl.BlockSpec((tb, _D), lambda i: (i, 0)),
            pl.BlockSpec((_D, _NSCORE), lambda i: (0, 0)),
        ],
        out_specs=pl.BlockSpec((_NSCORE, tb), lambda i: (0, i)),
        out_shape=jax.ShapeDtypeStruct((_NSCORE, _N), jnp.float32),
        compiler_params=pltpu.CompilerParams(
            dimension_semantics=("arbitrary",)),
    )(xf, wt)

    route = functools.partial(
        pl.kernel,
        out_type=[
            jax.ShapeDtypeStruct((_N * 6,), jnp.float32),
            jax.ShapeDtypeStruct((_N * 6,), jnp.int32),
            jax.ShapeDtypeStruct((_NW, _NE), jnp.float32),
        ],
        mesh=plsc.VectorSubcoreMesh(core_axis_name="c", subcore_axis_name="s"),
        scratch_types=[
            pltpu.VMEM((_NSCORE, _CHUNK), jnp.float32),
            pltpu.VMEM((_CHUNK * 6,), jnp.float32),
            pltpu.VMEM((_CHUNK * 6,), jnp.int32),
            pltpu.VMEM((16,), jnp.float32),
        ],
        compiler_params=pltpu.CompilerParams(needs_layout_passes=False),
    )(_route_body)
    rw_flat, idx_flat, counts = route(scores_t)

    aux = pl.pallas_call(
        _aux_body,
        out_shape=jax.ShapeDtypeStruct((1, 1), jnp.float32),
    )(counts)

    return (rw_flat.reshape(_B, _T, 6), idx_flat.reshape(_B, _T, 6),
            aux[0, 0])


def kernel(x, W_expert, W_group):
    return _run(x, W_expert, W_group)
